# Initial kernel scaffold; baseline (speedup 1.0000x reference)
import functools

import jax
import jax.numpy as jnp
from jax import lax
from jax.experimental import pallas as pl
from jax.experimental.pallas import tpu as pltpu
from jax.experimental.pallas import tpu_sc as plsc

N = 10000
E = 320000
NPAD = N + 16
NW = 32
EPW = E // NW
CHUNK = 128
NCH = 80
KW = NCH * CHUNK
ZROWS = NPAD // 16
WROWS = N // 16


def _make_spmm(cp):
    mesh = plsc.VectorSubcoreMesh(core_axis_name="c", subcore_axis_name="s")

    @functools.partial(
        pl.kernel,
        mesh=mesh,
        out_type=jax.ShapeDtypeStruct((2 * N, cp), jnp.float32),
        scratch_types=[
            pltpu.VMEM((NCH, CHUNK), jnp.int32),
            pltpu.VMEM((NCH, CHUNK), jnp.int32),
            pltpu.VMEM((CHUNK, cp), jnp.float32),
            pltpu.VMEM((CHUNK, cp), jnp.float32),
            pltpu.VMEM_SHARED((NPAD, cp), jnp.float32),
            pltpu.SemaphoreType.DMA,
            pltpu.SemaphoreType.DMA,
        ],
    )
    def spmm(ys_hbm, src_hbm, dst_hbm, out_hbm, src_v, dst_v, buf0, buf1,
             acc, sem0, sem1):
        c = lax.axis_index("c")
        s = lax.axis_index("s")
        w = c * 16 + s

        zero16 = jnp.zeros((16,), jnp.float32)

        def zrow(i, carry):
            for j in range(cp // 16):
                buf0[i, pl.ds(j * 16, 16)] = zero16
            return carry

        lax.fori_loop(0, CHUNK, zrow, 0)
        base = s * ZROWS
        for t in range(ZROWS // CHUNK):
            pltpu.sync_copy(buf0, acc.at[pl.ds(base + t * CHUNK, CHUNK)])
        rem = ZROWS % CHUNK
        if rem:
            pltpu.sync_copy(buf0.at[pl.ds(0, rem)],
                            acc.at[pl.ds(base + (ZROWS // CHUNK) * CHUNK, rem)])
        plsc.subcore_barrier()

        pltpu.sync_copy(src_hbm.at[w], src_v)
        pltpu.sync_copy(dst_hbm.at[w], dst_v)

        def start(j, buf, sem):
            pltpu.async_copy(ys_hbm.at[src_v.at[j]], buf, sem)

        def wait(j, buf, sem):
            pltpu.make_async_copy(ys_hbm.at[src_v.at[j]], buf, sem).wait()

        start(0, buf0, sem0)

        def body(g, carry):
            j0 = g * 2
            wait(j0, buf0, sem0)
            start(j0 + 1, buf1, sem1)
            pltpu.sync_copy(buf0, acc.at[dst_v.at[j0]], add=True)
            wait(j0 + 1, buf1, sem1)

            @pl.when(g + 1 < NCH // 2)
            def _():
                start(j0 + 2, buf0, sem0)

            pltpu.sync_copy(buf1, acc.at[dst_v.at[j0 + 1]], add=True)
            return carry

        lax.fori_loop(0, NCH // 2, body, 0)
        plsc.subcore_barrier()

        rb = s * WROWS
        pltpu.sync_copy(acc.at[pl.ds(rb, WROWS)],
                        out_hbm.at[pl.ds(c * N + rb, WROWS)])

    return spmm


_spmm16 = _make_spmm(16)
_spmm32 = _make_spmm(32)
_spmm112 = _make_spmm(112)


def _tc(body, out_shapes):
    return pl.pallas_call(
        body,
        out_shape=[jax.ShapeDtypeStruct(s, jnp.float32) for s in out_shapes],
    )


_DOT = functools.partial(jnp.dot, precision=lax.Precision.HIGHEST,
                         preferred_element_type=jnp.float32)


def _comb(Sref, lo, hi, dinv, y, b):
    S = Sref[0:N, lo:hi] + Sref[N:2 * N, lo:hi]
    return dinv * S + 2.0 * dinv * dinv * y + b


def _prep_body(degS, x, W1, W1_2, dinv_o, xw1_o, xw2_o, ys1_o, ys2_o):
    deg = degS[0:N, 0:1] + degS[N:2 * N, 0:1] + 2.0
    dinv = lax.rsqrt(deg)
    dinv_o[...] = dinv
    xw1 = _DOT(x[...], W1[...])
    xw2 = _DOT(x[...], W1_2[...])
    xw1_o[...] = xw1
    xw2_o[...] = xw2
    ys1_o[...] = dinv * xw1
    ys2_o[...] = dinv * xw2


def _dense2_body(S1a, S1b, xw1, xw2, dinv_r, W2, W2_2, b1, b1_2,
                 y2m_o, y2f_o, ys2_o):
    dinv = dinv_r[...]
    h1 = jax.nn.relu(_comb(S1a, 0, 100, dinv, xw1[...], b1[...]))
    h2 = jax.nn.relu(_comb(S1b, 0, 100, dinv, xw2[...], b1_2[...]))
    y2m = _DOT(h1, W2[...])
    y2f = _DOT(h2, W2_2[...])
    y2m_o[...] = y2m
    y2f_o[...] = y2f
    ys2_o[...] = dinv * jnp.concatenate([y2m, y2f], axis=1)


def _dense3_body(S2, y2m, y2f, dinv_r, m, f, b2, b2_2, W2m, W2f,
                 y3m_o, y3f_o, ys3_o):
    dinv = dinv_r[...]
    c2m = _comb(S2, 0, 1, dinv, y2m[...], b2[...])
    c2f = _comb(S2, 1, 2, dinv, y2f[...], b2_2[...])
    y3m = _DOT(jnp.concatenate([c2m, m[...]], axis=1), W2m[...])
    y3f = _DOT(jnp.concatenate([c2f, f[...]], axis=1), W2f[...])
    y3m_o[...] = y3m
    y3f_o[...] = y3f
    ys3_o[...] = dinv * jnp.concatenate([y3m, y3f], axis=1)


def _dense4_body(S3, y3m, y3f, dinv_r, b2m, b2f, W2m_1, W2f_1,
                 y4m_o, y4f_o, ys4_o):
    dinv = dinv_r[...]
    hm2 = jax.nn.relu(_comb(S3, 0, 10, dinv, y3m[...], b2m[...]))
    hf2 = jax.nn.relu(_comb(S3, 10, 20, dinv, y3f[...], b2f[...]))
    y4m = _DOT(hm2, W2m_1[...])
    y4f = _DOT(hf2, W2f_1[...])
    y4m_o[...] = y4m
    y4f_o[...] = y4f
    ys4_o[...] = dinv * jnp.concatenate([y4m, y4f], axis=1)


def _dense5_body(S4, y4m, y4f, dinv_r, b2m_1, b2f_1, WA,
                 hmbr_o, hfbr_o, y5_o, ys5_o):
    dinv = dinv_r[...]
    hm_br = _comb(S4, 0, 1, dinv, y4m[...], b2m_1[...])
    hf_br = _comb(S4, 1, 2, dinv, y4f[...], b2f_1[...])
    hmbr_o[...] = hm_br
    hfbr_o[...] = hf_br
    hcat = jnp.concatenate([jax.nn.relu(hm_br), jax.nn.relu(hf_br)], axis=1)
    y5 = _DOT(hcat, WA[...])
    y5_o[...] = y5
    ys5_o[...] = dinv * y5


def _dense6_body(S5, y5, dinv_r, bA, WA_1, y6_o, ys6_o):
    dinv = dinv_r[...]
    hA = jax.nn.relu(_comb(S5, 0, 10, dinv, y5[...], bA[...]))
    y6 = _DOT(hA, WA_1[...])
    y6_o[...] = y6
    ys6_o[...] = dinv * y6


def _dense7_body(S6, y6, dinv_r, bA_1, h_o):
    dinv = dinv_r[...]
    h_o[...] = _comb(S6, 0, 1, dinv, y6[...], bA_1[...])


def _pad_table(ys, cp):
    n, c = ys.shape
    return jnp.pad(ys, ((0, NPAD - n), (0, cp - c)))


def kernel(x, edge_index, edge_weight, m, f, W1, b1, W1_2, b1_2, W2, b2,
           W2_2, b2_2, W2m, b2m, W2m_1, b2m_1, W2f, b2f, W2f_1, b2f_1,
           WA, bA, WA_1, bA_1):
    pad_idx = N + (jnp.arange(KW - EPW, dtype=jnp.int32) % 16)
    pad_blk = jnp.broadcast_to(pad_idx, (NW, KW - EPW))
    srcw = jnp.concatenate([edge_index[0].reshape(NW, EPW), pad_blk], axis=1)
    dstw = jnp.concatenate([edge_index[1].reshape(NW, EPW), pad_blk], axis=1)
    srcw = srcw.reshape(NW, NCH, CHUNK)
    dstw = dstw.reshape(NW, NCH, CHUNK)

    b1r = b1.reshape(1, -1)
    b1_2r = b1_2.reshape(1, -1)
    b2r = b2.reshape(1, -1)
    b2_2r = b2_2.reshape(1, -1)
    b2mr = b2m.reshape(1, -1)
    b2fr = b2f.reshape(1, -1)
    b2m_1r = b2m_1.reshape(1, -1)
    b2f_1r = b2f_1.reshape(1, -1)
    bAr = bA.reshape(1, -1)
    bA_1r = bA_1.reshape(1, -1)

    ones_t = _pad_table(jnp.ones((N, 1), jnp.float32), 16)
    degS = _spmm16(ones_t, srcw, dstw)

    dinv, xw1, xw2, ys1, ys2 = _tc(
        _prep_body, [(N, 1), (N, 100), (N, 100), (N, 100), (N, 100)],
    )(degS, x, W1, W1_2)
    S1a = _spmm112(_pad_table(ys1, 112), srcw, dstw)
    S1b = _spmm112(_pad_table(ys2, 112), srcw, dstw)

    y2m, y2f, ys2c = _tc(
        _dense2_body, [(N, 1), (N, 1), (N, 2)],
    )(S1a, S1b, xw1, xw2, dinv, W2, W2_2, b1r, b1_2r)
    S2 = _spmm16(_pad_table(ys2c, 16), srcw, dstw)

    y3m, y3f, ys3 = _tc(
        _dense3_body, [(N, 10), (N, 10), (N, 20)],
    )(S2, y2m, y2f, dinv, m, f, b2r, b2_2r, W2m, W2f)
    S3 = _spmm32(_pad_table(ys3, 32), srcw, dstw)

    y4m, y4f, ys4 = _tc(
        _dense4_body, [(N, 1), (N, 1), (N, 2)],
    )(S3, y3m, y3f, dinv, b2mr, b2fr, W2m_1, W2f_1)
    S4 = _spmm16(_pad_table(ys4, 16), srcw, dstw)

    hm_br, hf_br, y5, ys5 = _tc(
        _dense5_body, [(N, 1), (N, 1), (N, 10), (N, 10)],
    )(S4, y4m, y4f, dinv, b2m_1r, b2f_1r, WA)
    S5 = _spmm16(_pad_table(ys5, 16), srcw, dstw)

    y6, ys6 = _tc(
        _dense6_body, [(N, 1), (N, 1)],
    )(S5, y5, dinv, bAr, WA_1)
    S6 = _spmm16(_pad_table(ys6, 16), srcw, dstw)

    (h,) = _tc(_dense7_body, [(N, 1)])(S6, y6, dinv, bA_1r)

    return (h, hm_br, hf_br)


# speedup vs baseline: 26.2119x; 26.2119x over previous
import functools

import jax
import jax.numpy as jnp
from jax import lax
from jax.experimental import pallas as pl
from jax.experimental.pallas import tpu as pltpu
from jax.experimental.pallas import tpu_sc as plsc

N = 10000
E = 320000
NPAD = 10240
NW = 32
EPW = E // NW
CHUNK = 128
NCH = 80
KW = NCH * CHUNK
ZROWS = NPAD // 16


def _make_spmm(cp):
    mesh = plsc.VectorSubcoreMesh(core_axis_name="c", subcore_axis_name="s")

    @functools.partial(
        pl.kernel,
        mesh=mesh,
        compiler_params=pltpu.CompilerParams(use_tc_tiling_on_sc=False),
        out_type=jax.ShapeDtypeStruct((2 * NPAD, cp), jnp.float32),
        scratch_types=[
            pltpu.VMEM((NCH, CHUNK), jnp.int32),
            pltpu.VMEM((NCH, CHUNK), jnp.int32),
            pltpu.VMEM((CHUNK, cp), jnp.float32),
            pltpu.VMEM((CHUNK, cp), jnp.float32),
            pltpu.VMEM_SHARED((NPAD, cp), jnp.float32),
            pltpu.SemaphoreType.DMA,
            pltpu.SemaphoreType.DMA,
        ],
    )
    def spmm(ys_hbm, src_hbm, dst_hbm, out_hbm, src_v, dst_v, buf0, buf1,
             acc, sem0, sem1):
        c = lax.axis_index("c")
        s = lax.axis_index("s")
        w = c * 16 + s

        zero16 = jnp.zeros((16,), jnp.float32)

        def zrow(i, carry):
            for j in range(cp // 16):
                buf0[i, pl.ds(j * 16, 16)] = zero16
            return carry

        lax.fori_loop(0, CHUNK, zrow, 0)
        base = s * ZROWS
        for t in range(ZROWS // CHUNK):
            pltpu.sync_copy(buf0, acc.at[pl.ds(base + t * CHUNK, CHUNK)])
        plsc.subcore_barrier()

        pltpu.sync_copy(src_hbm.at[w], src_v)
        pltpu.sync_copy(dst_hbm.at[w], dst_v)

        def start(j, buf, sem):
            pltpu.async_copy(ys_hbm.at[src_v.at[j]], buf, sem)

        def wait(j, buf, sem):
            pltpu.make_async_copy(ys_hbm.at[src_v.at[j]], buf, sem).wait()

        start(0, buf0, sem0)

        def body(g, carry):
            j0 = g * 2
            wait(j0, buf0, sem0)
            start(j0 + 1, buf1, sem1)
            pltpu.sync_copy(buf0, acc.at[dst_v.at[j0]], add=True)
            wait(j0 + 1, buf1, sem1)

            @pl.when(g + 1 < NCH // 2)
            def _():
                start(j0 + 2, buf0, sem0)

            pltpu.sync_copy(buf1, acc.at[dst_v.at[j0 + 1]], add=True)
            return carry

        lax.fori_loop(0, NCH // 2, body, 0)
        plsc.subcore_barrier()

        pltpu.sync_copy(acc.at[pl.ds(base, ZROWS)],
                        out_hbm.at[pl.ds(c * NPAD + base, ZROWS)])

    return spmm


_spmm16 = _make_spmm(16)
_spmm32 = _make_spmm(32)
_spmm112 = _make_spmm(112)


R = 2000


def _tc(body, in_kinds, out_widths):
    in_specs = []
    for kind, p in in_kinds:
        if kind == "S":
            in_specs.append(pl.BlockSpec((2, R, p), lambda i: (0, i, 0)))
        elif kind == "r":
            in_specs.append(pl.BlockSpec((R, p), lambda i: (i, 0)))
        else:
            in_specs.append(pl.BlockSpec(p, lambda i: (0,) * len(p)))
    return pl.pallas_call(
        body,
        grid=(N // R,),
        in_specs=in_specs,
        out_specs=[pl.BlockSpec((R, w), lambda i: (i, 0))
                   for w in out_widths],
        out_shape=[jax.ShapeDtypeStruct((N, w), jnp.float32)
                   for w in out_widths],
    )


_DOT = functools.partial(jnp.dot, precision=lax.Precision.HIGHEST,
                         preferred_element_type=jnp.float32)


def _comb(Sref, lo, hi, dinv, y, b):
    S = Sref[0, :, lo:hi] + Sref[1, :, lo:hi]
    return dinv * S + 2.0 * dinv * dinv * y + b


def _prep_body(degS, x, W1, W1_2, dinv_o, xw1_o, xw2_o, ys1_o, ys2_o):
    deg = degS[0, :, 0:1] + degS[1, :, 0:1] + 2.0
    dinv = lax.rsqrt(deg)
    dinv_o[...] = dinv
    xw1 = _DOT(x[...], W1[...])
    xw2 = _DOT(x[...], W1_2[...])
    xw1_o[...] = xw1
    xw2_o[...] = xw2
    ys1_o[...] = dinv * xw1
    ys2_o[...] = dinv * xw2


def _dense2_body(S1a, S1b, xw1, xw2, dinv_r, W2, W2_2, b1, b1_2,
                 y2m_o, y2f_o, ys2_o):
    dinv = dinv_r[...]
    h1 = jax.nn.relu(_comb(S1a, 0, 100, dinv, xw1[...], b1[...]))
    h2 = jax.nn.relu(_comb(S1b, 0, 100, dinv, xw2[...], b1_2[...]))
    y2m = _DOT(h1, W2[...])
    y2f = _DOT(h2, W2_2[...])
    y2m_o[...] = y2m
    y2f_o[...] = y2f
    ys2_o[...] = dinv * jnp.concatenate([y2m, y2f], axis=1)


def _dense3_body(S2, y2m, y2f, dinv_r, m, f, b2, b2_2, W2m, W2f,
                 y3m_o, y3f_o, ys3_o):
    dinv = dinv_r[...]
    c2m = _comb(S2, 0, 1, dinv, y2m[...], b2[...])
    c2f = _comb(S2, 1, 2, dinv, y2f[...], b2_2[...])
    y3m = _DOT(jnp.concatenate([c2m, m[...]], axis=1), W2m[...])
    y3f = _DOT(jnp.concatenate([c2f, f[...]], axis=1), W2f[...])
    y3m_o[...] = y3m
    y3f_o[...] = y3f
    ys3_o[...] = dinv * jnp.concatenate([y3m, y3f], axis=1)


def _dense4_body(S3, y3m, y3f, dinv_r, b2m, b2f, W2m_1, W2f_1,
                 y4m_o, y4f_o, ys4_o):
    dinv = dinv_r[...]
    hm2 = jax.nn.relu(_comb(S3, 0, 10, dinv, y3m[...], b2m[...]))
    hf2 = jax.nn.relu(_comb(S3, 10, 20, dinv, y3f[...], b2f[...]))
    y4m = _DOT(hm2, W2m_1[...])
    y4f = _DOT(hf2, W2f_1[...])
    y4m_o[...] = y4m
    y4f_o[...] = y4f
    ys4_o[...] = dinv * jnp.concatenate([y4m, y4f], axis=1)


def _dense5_body(S4, y4m, y4f, dinv_r, b2m_1, b2f_1, WA,
                 hmbr_o, hfbr_o, y5_o, ys5_o):
    dinv = dinv_r[...]
    hm_br = _comb(S4, 0, 1, dinv, y4m[...], b2m_1[...])
    hf_br = _comb(S4, 1, 2, dinv, y4f[...], b2f_1[...])
    hmbr_o[...] = hm_br
    hfbr_o[...] = hf_br
    hcat = jnp.concatenate([jax.nn.relu(hm_br), jax.nn.relu(hf_br)], axis=1)
    y5 = _DOT(hcat, WA[...])
    y5_o[...] = y5
    ys5_o[...] = dinv * y5


def _dense6_body(S5, y5, dinv_r, bA, WA_1, y6_o, ys6_o):
    dinv = dinv_r[...]
    hA = jax.nn.relu(_comb(S5, 0, 10, dinv, y5[...], bA[...]))
    y6 = _DOT(hA, WA_1[...])
    y6_o[...] = y6
    ys6_o[...] = dinv * y6


def _dense7_body(S6, y6, dinv_r, bA_1, h_o):
    dinv = dinv_r[...]
    h_o[...] = _comb(S6, 0, 1, dinv, y6[...], bA_1[...])


def _pad_table(ys, cp):
    n, c = ys.shape
    return jnp.pad(ys, ((0, NPAD - n), (0, cp - c)))


def kernel(x, edge_index, edge_weight, m, f, W1, b1, W1_2, b1_2, W2, b2,
           W2_2, b2_2, W2m, b2m, W2m_1, b2m_1, W2f, b2f, W2f_1, b2f_1,
           WA, bA, WA_1, bA_1):
    pad_idx = N + (jnp.arange(KW - EPW, dtype=jnp.int32) % 16)
    pad_blk = jnp.broadcast_to(pad_idx, (NW, KW - EPW))
    srcw = jnp.concatenate([edge_index[0].reshape(NW, EPW), pad_blk], axis=1)
    dstw = jnp.concatenate([edge_index[1].reshape(NW, EPW), pad_blk], axis=1)
    srcw = srcw.reshape(NW, NCH, CHUNK)
    dstw = dstw.reshape(NW, NCH, CHUNK)

    b1r = b1.reshape(1, -1)
    b1_2r = b1_2.reshape(1, -1)
    b2r = b2.reshape(1, -1)
    b2_2r = b2_2.reshape(1, -1)
    b2mr = b2m.reshape(1, -1)
    b2fr = b2f.reshape(1, -1)
    b2m_1r = b2m_1.reshape(1, -1)
    b2f_1r = b2f_1.reshape(1, -1)
    bAr = bA.reshape(1, -1)
    bA_1r = bA_1.reshape(1, -1)

    ones_t = _pad_table(jnp.ones((N, 1), jnp.float32), 16)
    degS = _spmm16(ones_t, srcw, dstw).reshape(2, NPAD, 16)

    dinv, xw1, xw2, ys1, ys2 = _tc(
        _prep_body,
        [("S", 16), ("r", 128), ("w", (128, 100)), ("w", (128, 100))],
        [1, 100, 100, 100, 100],
    )(degS, x, W1, W1_2)
    S1a = _spmm112(_pad_table(ys1, 112), srcw, dstw).reshape(2, NPAD, 112)
    S1b = _spmm112(_pad_table(ys2, 112), srcw, dstw).reshape(2, NPAD, 112)

    y2m, y2f, ys2c = _tc(
        _dense2_body,
        [("S", 112), ("S", 112), ("r", 100), ("r", 100), ("r", 1),
         ("w", (100, 1)), ("w", (100, 1)), ("w", (1, 100)), ("w", (1, 100))],
        [1, 1, 2],
    )(S1a, S1b, xw1, xw2, dinv, W2, W2_2, b1r, b1_2r)
    S2 = _spmm16(_pad_table(ys2c, 16), srcw, dstw).reshape(2, NPAD, 16)

    y3m, y3f, ys3 = _tc(
        _dense3_body,
        [("S", 16), ("r", 1), ("r", 1), ("r", 1), ("r", 1), ("r", 1),
         ("w", (1, 1)), ("w", (1, 1)), ("w", (2, 10)), ("w", (2, 10))],
        [10, 10, 20],
    )(S2, y2m, y2f, dinv, m, f, b2r, b2_2r, W2m, W2f)
    S3 = _spmm32(_pad_table(ys3, 32), srcw, dstw).reshape(2, NPAD, 32)

    y4m, y4f, ys4 = _tc(
        _dense4_body,
        [("S", 32), ("r", 10), ("r", 10), ("r", 1),
         ("w", (1, 10)), ("w", (1, 10)), ("w", (10, 1)), ("w", (10, 1))],
        [1, 1, 2],
    )(S3, y3m, y3f, dinv, b2mr, b2fr, W2m_1, W2f_1)
    S4 = _spmm16(_pad_table(ys4, 16), srcw, dstw).reshape(2, NPAD, 16)

    hm_br, hf_br, y5, ys5 = _tc(
        _dense5_body,
        [("S", 16), ("r", 1), ("r", 1), ("r", 1),
         ("w", (1, 1)), ("w", (1, 1)), ("w", (2, 10))],
        [1, 1, 10, 10],
    )(S4, y4m, y4f, dinv, b2m_1r, b2f_1r, WA)
    S5 = _spmm16(_pad_table(ys5, 16), srcw, dstw).reshape(2, NPAD, 16)

    y6, ys6 = _tc(
        _dense6_body,
        [("S", 16), ("r", 10), ("r", 1), ("w", (1, 10)), ("w", (10, 1))],
        [1, 1],
    )(S5, y5, dinv, bAr, WA_1)
    S6 = _spmm16(_pad_table(ys6, 16), srcw, dstw).reshape(2, NPAD, 16)

    (h,) = _tc(
        _dense7_body,
        [("S", 16), ("r", 1), ("r", 1), ("w", (1, 1))],
        [1],
    )(S6, y6, dinv, bA_1r)

    return (h, hm_br, hf_br)


# speedup vs baseline: 40.0713x; 1.5287x over previous
import functools

import jax
import jax.numpy as jnp
from jax import lax
from jax.experimental import pallas as pl
from jax.experimental.pallas import tpu as pltpu
from jax.experimental.pallas import tpu_sc as plsc

N = 10000
E = 320000
NPAD = 10240
NW = 32
EPW = E // NW
CHUNK = 128
NCH = 80
KW = NCH * CHUNK
ZROWS = NPAD // 16
NBUF = 5
D = 3


def _make_spmm(cp):
    mesh = plsc.VectorSubcoreMesh(core_axis_name="c", subcore_axis_name="s")

    @functools.partial(
        pl.kernel,
        mesh=mesh,
        compiler_params=pltpu.CompilerParams(use_tc_tiling_on_sc=False),
        out_type=jax.ShapeDtypeStruct((2 * NPAD, cp), jnp.float32),
        scratch_types=[
            pltpu.VMEM((NCH, CHUNK), jnp.int32),
            pltpu.VMEM((NCH, CHUNK), jnp.int32),
            [pltpu.VMEM((CHUNK, cp), jnp.float32)] * NBUF,
            pltpu.VMEM_SHARED((NPAD, cp), jnp.float32),
            [pltpu.SemaphoreType.DMA] * NBUF,
            [pltpu.SemaphoreType.DMA] * NBUF,
        ],
    )
    def spmm(ys_hbm, src_hbm, dst_hbm, out_hbm, src_v, dst_v, bufs,
             acc, gsems, ssems):
        c = lax.axis_index("c")
        s = lax.axis_index("s")
        w = c * 16 + s
        base = s * ZROWS

        pltpu.async_copy(src_hbm.at[w], src_v, ssems[0])
        pltpu.async_copy(dst_hbm.at[w], dst_v, ssems[1])

        zero16 = jnp.zeros((16,), jnp.float32)

        def zrow(i, carry):
            for j in range(cp // 16):
                bufs[0][i, pl.ds(j * 16, 16)] = zero16
            return carry

        lax.fori_loop(0, CHUNK, zrow, 0)
        for t in range(ZROWS // CHUNK):
            pltpu.async_copy(bufs[0],
                             acc.at[pl.ds(base + t * CHUNK, CHUNK)],
                             gsems[t % NBUF])
        for t in range(ZROWS // CHUNK):
            pltpu.make_async_copy(bufs[0],
                                  acc.at[pl.ds(base + t * CHUNK, CHUNK)],
                                  gsems[t % NBUF]).wait()
        pltpu.make_async_copy(src_hbm.at[w], src_v, ssems[0]).wait()
        pltpu.make_async_copy(dst_hbm.at[w], dst_v, ssems[1]).wait()
        plsc.subcore_barrier()

        def g_start(j, b):
            pltpu.async_copy(ys_hbm.at[src_v.at[j]], bufs[b], gsems[b])

        def g_wait(j, b):
            pltpu.make_async_copy(ys_hbm.at[src_v.at[j]], bufs[b],
                                  gsems[b]).wait()

        def s_start(j, b):
            pltpu.async_copy(bufs[b], acc.at[dst_v.at[j]], ssems[b],
                             add=True)

        def s_wait(j, b):
            pltpu.make_async_copy(bufs[b], acc.at[dst_v.at[j]],
                                  ssems[b]).wait()

        for b in range(D):
            g_start(b, b)

        def outer(g, carry):
            j0 = g * NBUF
            for u in range(NBUF):
                j = j0 + u
                g_wait(j, u)
                s_start(j, u)
                k = j + D
                bk = (u + D) % NBUF

                @pl.when(k < NCH)
                def _():
                    @pl.when(k >= NBUF)
                    def _():
                        s_wait(k - NBUF, bk)

                    g_start(k, bk)

            return carry

        lax.fori_loop(0, NCH // NBUF, outer, 0)
        for u in range(NBUF):
            s_wait(NCH - NBUF + u, u)
        plsc.subcore_barrier()

        pltpu.sync_copy(acc.at[pl.ds(base, ZROWS)],
                        out_hbm.at[pl.ds(c * NPAD + base, ZROWS)])

    return spmm


_spmm16 = _make_spmm(16)
_spmm32 = _make_spmm(32)
_spmm64 = _make_spmm(64)


R = 2000


def _tc(body, in_kinds, out_widths):
    in_specs = []
    for kind, p in in_kinds:
        if kind == "S":
            in_specs.append(pl.BlockSpec((2, R, p), lambda i: (0, i, 0)))
        elif kind == "r":
            in_specs.append(pl.BlockSpec((R, p), lambda i: (i, 0)))
        else:
            in_specs.append(pl.BlockSpec(p, lambda i: (0,) * len(p)))
    return pl.pallas_call(
        body,
        grid=(N // R,),
        in_specs=in_specs,
        out_specs=[pl.BlockSpec((R, w), lambda i: (i, 0))
                   for w in out_widths],
        out_shape=[jax.ShapeDtypeStruct((N, w), jnp.float32)
                   for w in out_widths],
    )


_DOT = functools.partial(jnp.dot, precision=lax.Precision.HIGHEST,
                         preferred_element_type=jnp.float32)


def _comb(Sref, lo, hi, dinv, y, b):
    S = Sref[0, :, lo:hi] + Sref[1, :, lo:hi]
    return dinv * S + 2.0 * dinv * dinv * y + b


def _prep_body(degS, x, dinv_o, ysx_o):
    deg = degS[0, :, 0:1] + degS[1, :, 0:1] + 2.0
    dinv = lax.rsqrt(deg)
    dinv_o[...] = dinv
    ysx_o[...] = dinv * x[...]


def _dense12_body(SxA, SxB, x, dinv_r, W1, W1_2, b1, b1_2, W2, W2_2,
                  y2m_o, y2f_o, ys2_o):
    dinv = dinv_r[...]
    u = jnp.concatenate(
        [_comb(SxA, 0, 64, dinv, x[:, 0:64], 0.0),
         _comb(SxB, 0, 64, dinv, x[:, 64:128], 0.0)], axis=1)
    h1 = jax.nn.relu(_DOT(u, W1[...]) + b1[...])
    h2 = jax.nn.relu(_DOT(u, W1_2[...]) + b1_2[...])
    y2m = _DOT(h1, W2[...])
    y2f = _DOT(h2, W2_2[...])
    y2m_o[...] = y2m
    y2f_o[...] = y2f
    ys2_o[...] = dinv * jnp.concatenate([y2m, y2f], axis=1)


def _dense3_body(S2, y2m, y2f, dinv_r, m, f, b2, b2_2, W2m, W2f,
                 y3m_o, y3f_o, ys3_o):
    dinv = dinv_r[...]
    c2m = _comb(S2, 0, 1, dinv, y2m[...], b2[...])
    c2f = _comb(S2, 1, 2, dinv, y2f[...], b2_2[...])
    y3m = _DOT(jnp.concatenate([c2m, m[...]], axis=1), W2m[...])
    y3f = _DOT(jnp.concatenate([c2f, f[...]], axis=1), W2f[...])
    y3m_o[...] = y3m
    y3f_o[...] = y3f
    ys3_o[...] = dinv * jnp.concatenate([y3m, y3f], axis=1)


def _dense4_body(S3, y3m, y3f, dinv_r, b2m, b2f, W2m_1, W2f_1,
                 y4m_o, y4f_o, ys4_o):
    dinv = dinv_r[...]
    hm2 = jax.nn.relu(_comb(S3, 0, 10, dinv, y3m[...], b2m[...]))
    hf2 = jax.nn.relu(_comb(S3, 10, 20, dinv, y3f[...], b2f[...]))
    y4m = _DOT(hm2, W2m_1[...])
    y4f = _DOT(hf2, W2f_1[...])
    y4m_o[...] = y4m
    y4f_o[...] = y4f
    ys4_o[...] = dinv * jnp.concatenate([y4m, y4f], axis=1)


def _dense5_body(S4, y4m, y4f, dinv_r, b2m_1, b2f_1, WA,
                 hmbr_o, hfbr_o, y5_o, ys5_o):
    dinv = dinv_r[...]
    hm_br = _comb(S4, 0, 1, dinv, y4m[...], b2m_1[...])
    hf_br = _comb(S4, 1, 2, dinv, y4f[...], b2f_1[...])
    hmbr_o[...] = hm_br
    hfbr_o[...] = hf_br
    hcat = jnp.concatenate([jax.nn.relu(hm_br), jax.nn.relu(hf_br)], axis=1)
    y5 = _DOT(hcat, WA[...])
    y5_o[...] = y5
    ys5_o[...] = dinv * y5


def _dense6_body(S5, y5, dinv_r, bA, WA_1, y6_o, ys6_o):
    dinv = dinv_r[...]
    hA = jax.nn.relu(_comb(S5, 0, 10, dinv, y5[...], bA[...]))
    y6 = _DOT(hA, WA_1[...])
    y6_o[...] = y6
    ys6_o[...] = dinv * y6


def _dense7_body(S6, y6, dinv_r, bA_1, h_o):
    dinv = dinv_r[...]
    h_o[...] = _comb(S6, 0, 1, dinv, y6[...], bA_1[...])


def _pad_table(ys, cp):
    n, c = ys.shape
    return jnp.pad(ys, ((0, NPAD - n), (0, cp - c)))


def kernel(x, edge_index, edge_weight, m, f, W1, b1, W1_2, b1_2, W2, b2,
           W2_2, b2_2, W2m, b2m, W2m_1, b2m_1, W2f, b2f, W2f_1, b2f_1,
           WA, bA, WA_1, bA_1):
    pad_idx = N + (jnp.arange(KW - EPW, dtype=jnp.int32) % 16)
    pad_blk = jnp.broadcast_to(pad_idx, (NW, KW - EPW))
    srcw = jnp.concatenate([edge_index[0].reshape(NW, EPW), pad_blk], axis=1)
    dstw = jnp.concatenate([edge_index[1].reshape(NW, EPW), pad_blk], axis=1)
    srcw = srcw.reshape(NW, NCH, CHUNK)
    dstw = dstw.reshape(NW, NCH, CHUNK)

    b1r = b1.reshape(1, -1)
    b1_2r = b1_2.reshape(1, -1)
    b2r = b2.reshape(1, -1)
    b2_2r = b2_2.reshape(1, -1)
    b2mr = b2m.reshape(1, -1)
    b2fr = b2f.reshape(1, -1)
    b2m_1r = b2m_1.reshape(1, -1)
    b2f_1r = b2f_1.reshape(1, -1)
    bAr = bA.reshape(1, -1)
    bA_1r = bA_1.reshape(1, -1)

    ones_t = _pad_table(jnp.ones((N, 1), jnp.float32), 16)
    degS = _spmm16(ones_t, srcw, dstw).reshape(2, NPAD, 16)

    dinv, ysx = _tc(
        _prep_body,
        [("S", 16), ("r", 128)],
        [1, 128],
    )(degS, x)
    SxA = _spmm64(_pad_table(ysx[:, 0:64], 64), srcw, dstw)
    SxB = _spmm64(_pad_table(ysx[:, 64:128], 64), srcw, dstw)
    SxA = SxA.reshape(2, NPAD, 64)
    SxB = SxB.reshape(2, NPAD, 64)

    y2m, y2f, ys2c = _tc(
        _dense12_body,
        [("S", 64), ("S", 64), ("r", 128), ("r", 1),
         ("w", (128, 100)), ("w", (128, 100)), ("w", (1, 100)),
         ("w", (1, 100)), ("w", (100, 1)), ("w", (100, 1))],
        [1, 1, 2],
    )(SxA, SxB, x, dinv, W1, W1_2, b1r, b1_2r, W2, W2_2)
    S2 = _spmm16(_pad_table(ys2c, 16), srcw, dstw).reshape(2, NPAD, 16)

    y3m, y3f, ys3 = _tc(
        _dense3_body,
        [("S", 16), ("r", 1), ("r", 1), ("r", 1), ("r", 1), ("r", 1),
         ("w", (1, 1)), ("w", (1, 1)), ("w", (2, 10)), ("w", (2, 10))],
        [10, 10, 20],
    )(S2, y2m, y2f, dinv, m, f, b2r, b2_2r, W2m, W2f)
    S3 = _spmm32(_pad_table(ys3, 32), srcw, dstw).reshape(2, NPAD, 32)

    y4m, y4f, ys4 = _tc(
        _dense4_body,
        [("S", 32), ("r", 10), ("r", 10), ("r", 1),
         ("w", (1, 10)), ("w", (1, 10)), ("w", (10, 1)), ("w", (10, 1))],
        [1, 1, 2],
    )(S3, y3m, y3f, dinv, b2mr, b2fr, W2m_1, W2f_1)
    S4 = _spmm16(_pad_table(ys4, 16), srcw, dstw).reshape(2, NPAD, 16)

    hm_br, hf_br, y5, ys5 = _tc(
        _dense5_body,
        [("S", 16), ("r", 1), ("r", 1), ("r", 1),
         ("w", (1, 1)), ("w", (1, 1)), ("w", (2, 10))],
        [1, 1, 10, 10],
    )(S4, y4m, y4f, dinv, b2m_1r, b2f_1r, WA)
    S5 = _spmm16(_pad_table(ys5, 16), srcw, dstw).reshape(2, NPAD, 16)

    y6, ys6 = _tc(
        _dense6_body,
        [("S", 16), ("r", 10), ("r", 1), ("w", (1, 10)), ("w", (10, 1))],
        [1, 1],
    )(S5, y5, dinv, bAr, WA_1)
    S6 = _spmm16(_pad_table(ys6, 16), srcw, dstw).reshape(2, NPAD, 16)

    (h,) = _tc(
        _dense7_body,
        [("S", 16), ("r", 1), ("r", 1), ("w", (1, 1))],
        [1],
    )(S6, y6, dinv, bA_1r)

    return (h, hm_br, hf_br)


# speedup vs baseline: 40.9613x; 1.0222x over previous
import functools

import jax
import jax.numpy as jnp
from jax import lax
from jax.experimental import pallas as pl
from jax.experimental.pallas import tpu as pltpu
from jax.experimental.pallas import tpu_sc as plsc

N = 10000
E = 320000
NPAD = 10240
NW = 32
EPW = E // NW
CHUNK = 128
NCH = 80
KW = NCH * CHUNK
ZROWS = NPAD // 16
NBUF = 5
D = 3


def _make_spmm(cp):
    mesh = plsc.VectorSubcoreMesh(core_axis_name="c", subcore_axis_name="s")

    @functools.partial(
        pl.kernel,
        mesh=mesh,
        compiler_params=pltpu.CompilerParams(use_tc_tiling_on_sc=False),
        out_type=jax.ShapeDtypeStruct((2 * NPAD, cp), jnp.float32),
        scratch_types=[
            pltpu.VMEM((NCH, CHUNK), jnp.int32),
            pltpu.VMEM((NCH, CHUNK), jnp.int32),
            [pltpu.VMEM((CHUNK, cp), jnp.float32)] * NBUF,
            pltpu.VMEM_SHARED((NPAD, cp), jnp.float32),
            [pltpu.SemaphoreType.DMA] * NBUF,
            [pltpu.SemaphoreType.DMA] * NBUF,
        ],
    )
    def spmm(ys_hbm, src_hbm, dst_hbm, out_hbm, src_v, dst_v, bufs,
             acc, gsems, ssems):
        c = lax.axis_index("c")
        s = lax.axis_index("s")
        w = c * 16 + s
        base = s * ZROWS

        pltpu.async_copy(src_hbm.at[w], src_v, ssems[0])
        pltpu.async_copy(dst_hbm.at[w], dst_v, ssems[1])

        zero16 = jnp.zeros((16,), jnp.float32)

        def zrow(i, carry):
            for j in range(cp // 16):
                bufs[0][i, pl.ds(j * 16, 16)] = zero16
            return carry

        lax.fori_loop(0, CHUNK, zrow, 0)
        for t in range(ZROWS // CHUNK):
            pltpu.async_copy(bufs[0],
                             acc.at[pl.ds(base + t * CHUNK, CHUNK)],
                             gsems[t % NBUF])
        for t in range(ZROWS // CHUNK):
            pltpu.make_async_copy(bufs[0],
                                  acc.at[pl.ds(base + t * CHUNK, CHUNK)],
                                  gsems[t % NBUF]).wait()
        pltpu.make_async_copy(src_hbm.at[w], src_v, ssems[0]).wait()
        pltpu.make_async_copy(dst_hbm.at[w], dst_v, ssems[1]).wait()
        plsc.subcore_barrier()

        def g_start(j, b):
            pltpu.async_copy(ys_hbm.at[src_v.at[j]], bufs[b], gsems[b])

        def g_wait(j, b):
            pltpu.make_async_copy(ys_hbm.at[src_v.at[j]], bufs[b],
                                  gsems[b]).wait()

        def s_start(j, b):
            pltpu.async_copy(bufs[b], acc.at[dst_v.at[j]], ssems[b],
                             add=True)

        def s_wait(j, b):
            pltpu.make_async_copy(bufs[b], acc.at[dst_v.at[j]],
                                  ssems[b]).wait()

        for b in range(D):
            g_start(b, b)

        def outer(g, carry):
            j0 = g * NBUF
            for u in range(NBUF):
                j = j0 + u
                g_wait(j, u)
                s_start(j, u)
                k = j + D
                bk = (u + D) % NBUF

                @pl.when(k < NCH)
                def _():
                    @pl.when(k >= NBUF)
                    def _():
                        s_wait(k - NBUF, bk)

                    g_start(k, bk)

            return carry

        lax.fori_loop(0, NCH // NBUF, outer, 0)
        for u in range(NBUF):
            s_wait(NCH - NBUF + u, u)
        plsc.subcore_barrier()

        pltpu.sync_copy(acc.at[pl.ds(base, ZROWS)],
                        out_hbm.at[pl.ds(c * NPAD + base, ZROWS)])

    return spmm


_spmm16 = _make_spmm(16)
_spmm64 = _make_spmm(64)


def _make_deg():
    mesh = plsc.VectorSubcoreMesh(core_axis_name="c", subcore_axis_name="s")

    @functools.partial(
        pl.kernel,
        mesh=mesh,
        compiler_params=pltpu.CompilerParams(use_tc_tiling_on_sc=False),
        out_type=jax.ShapeDtypeStruct((2 * NPAD, 16), jnp.float32),
        scratch_types=[
            pltpu.VMEM((NCH, CHUNK), jnp.int32),
            pltpu.VMEM((CHUNK, 16), jnp.float32),
            pltpu.VMEM((CHUNK, 16), jnp.float32),
            pltpu.VMEM_SHARED((NPAD, 16), jnp.float32),
            [pltpu.SemaphoreType.DMA] * NBUF,
            pltpu.SemaphoreType.DMA,
        ],
    )
    def deg(dst_hbm, out_hbm, dst_v, ones_b, zero_b, acc, ssems, isem):
        c = lax.axis_index("c")
        s = lax.axis_index("s")
        w = c * 16 + s
        base = s * ZROWS

        pltpu.async_copy(dst_hbm.at[w], dst_v, isem)

        one16 = jnp.ones((16,), jnp.float32)
        zero16 = jnp.zeros((16,), jnp.float32)

        def frow(i, carry):
            ones_b[i, pl.ds(0, 16)] = one16
            zero_b[i, pl.ds(0, 16)] = zero16
            return carry

        lax.fori_loop(0, CHUNK, frow, 0)
        for t in range(ZROWS // CHUNK):
            pltpu.async_copy(zero_b, acc.at[pl.ds(base + t * CHUNK, CHUNK)],
                             ssems[t % NBUF])
        for t in range(ZROWS // CHUNK):
            pltpu.make_async_copy(zero_b,
                                  acc.at[pl.ds(base + t * CHUNK, CHUNK)],
                                  ssems[t % NBUF]).wait()
        pltpu.make_async_copy(dst_hbm.at[w], dst_v, isem).wait()
        plsc.subcore_barrier()

        def s_start(j, u):
            pltpu.async_copy(ones_b, acc.at[dst_v.at[j]], ssems[u], add=True)

        def s_wait(j, u):
            pltpu.make_async_copy(ones_b, acc.at[dst_v.at[j]],
                                  ssems[u]).wait()

        def outer(g, carry):
            j0 = g * NBUF
            for u in range(NBUF):
                j = j0 + u

                @pl.when(j >= NBUF)
                def _():
                    s_wait(j - NBUF, u)

                s_start(j, u)
            return carry

        lax.fori_loop(0, NCH // NBUF, outer, 0)
        for u in range(NBUF):
            s_wait(NCH - NBUF + u, u)
        plsc.subcore_barrier()

        pltpu.sync_copy(acc.at[pl.ds(base, ZROWS)],
                        out_hbm.at[pl.ds(c * NPAD + base, ZROWS)])

    return deg


_deg = _make_deg()


R = 2000


def _tc(body, in_kinds, out_widths):
    in_specs = []
    for kind, p in in_kinds:
        if kind == "S":
            in_specs.append(pl.BlockSpec((2, R, p), lambda i: (0, i, 0)))
        elif kind == "r":
            in_specs.append(pl.BlockSpec((R, p), lambda i: (i, 0)))
        else:
            in_specs.append(pl.BlockSpec(p, lambda i: (0,) * len(p)))
    return pl.pallas_call(
        body,
        grid=(N // R,),
        in_specs=in_specs,
        out_specs=[pl.BlockSpec((R, w), lambda i: (i, 0))
                   for w in out_widths],
        out_shape=[jax.ShapeDtypeStruct((N, w), jnp.float32)
                   for w in out_widths],
    )


_DOT = functools.partial(jnp.dot, precision=lax.Precision.HIGHEST,
                         preferred_element_type=jnp.float32)


def _comb(Sref, lo, hi, dinv, y, b):
    S = Sref[0, :, lo:hi] + Sref[1, :, lo:hi]
    return dinv * S + 2.0 * dinv * dinv * y + b


def _prep_body(degS, x, dinv_o, ysx_o):
    deg = degS[0, :, 0:1] + degS[1, :, 0:1] + 2.0
    dinv = lax.rsqrt(deg)
    dinv_o[...] = dinv
    ysx_o[...] = dinv * x[...]


def _dense12_body(SxA, SxB, x, dinv_r, W1, W1_2, b1, b1_2, W2, W2_2,
                  y2m_o, y2f_o, ys2_o):
    dinv = dinv_r[...]
    u = jnp.concatenate(
        [_comb(SxA, 0, 64, dinv, x[:, 0:64], 0.0),
         _comb(SxB, 0, 64, dinv, x[:, 64:128], 0.0)], axis=1)
    h1 = jax.nn.relu(_DOT(u, W1[...]) + b1[...])
    h2 = jax.nn.relu(_DOT(u, W1_2[...]) + b1_2[...])
    y2m = _DOT(h1, W2[...])
    y2f = _DOT(h2, W2_2[...])
    y2m_o[...] = y2m
    y2f_o[...] = y2f
    ys2_o[...] = dinv * jnp.concatenate([y2m, y2f], axis=1)


def _dense3_body(S2, y2m, y2f, dinv_r, m, f, b2, b2_2,
                 c2m_o, c2f_o, ys3_o):
    dinv = dinv_r[...]
    c2m = _comb(S2, 0, 1, dinv, y2m[...], b2[...])
    c2f = _comb(S2, 1, 2, dinv, y2f[...], b2_2[...])
    c2m_o[...] = c2m
    c2f_o[...] = c2f
    ys3_o[...] = dinv * jnp.concatenate([c2m, c2f, m[...], f[...]], axis=1)


def _dense4_body(S3, c2m, c2f, m, f, dinv_r, W2m, W2f, b2m, b2f,
                 W2m_1, W2f_1, y4m_o, y4f_o, ys4_o):
    dinv = dinv_r[...]
    t_c2m = S3[0, :, 0:1] + S3[1, :, 0:1]
    t_c2f = S3[0, :, 1:2] + S3[1, :, 1:2]
    t_m = S3[0, :, 2:3] + S3[1, :, 2:3]
    t_f = S3[0, :, 3:4] + S3[1, :, 3:4]
    S3m = _DOT(t_c2m, W2m[0:1, :]) + _DOT(t_m, W2m[1:2, :])
    S3f = _DOT(t_c2f, W2f[0:1, :]) + _DOT(t_f, W2f[1:2, :])
    y3m = _DOT(c2m[...], W2m[0:1, :]) + _DOT(m[...], W2m[1:2, :])
    y3f = _DOT(c2f[...], W2f[0:1, :]) + _DOT(f[...], W2f[1:2, :])
    d2 = 2.0 * dinv * dinv
    hm2 = jax.nn.relu(dinv * S3m + d2 * y3m + b2m[...])
    hf2 = jax.nn.relu(dinv * S3f + d2 * y3f + b2f[...])
    y4m = _DOT(hm2, W2m_1[...])
    y4f = _DOT(hf2, W2f_1[...])
    y4m_o[...] = y4m
    y4f_o[...] = y4f
    ys4_o[...] = dinv * jnp.concatenate([y4m, y4f], axis=1)


def _dense5_body(S4, y4m, y4f, dinv_r, b2m_1, b2f_1, WA,
                 hmbr_o, hfbr_o, y5_o, ys5_o):
    dinv = dinv_r[...]
    hm_br = _comb(S4, 0, 1, dinv, y4m[...], b2m_1[...])
    hf_br = _comb(S4, 1, 2, dinv, y4f[...], b2f_1[...])
    hmbr_o[...] = hm_br
    hfbr_o[...] = hf_br
    hcat = jnp.concatenate([jax.nn.relu(hm_br), jax.nn.relu(hf_br)], axis=1)
    y5 = _DOT(hcat, WA[...])
    y5_o[...] = y5
    ys5_o[...] = dinv * y5


def _dense6_body(S5, y5, dinv_r, bA, WA_1, y6_o, ys6_o):
    dinv = dinv_r[...]
    hA = jax.nn.relu(_comb(S5, 0, 10, dinv, y5[...], bA[...]))
    y6 = _DOT(hA, WA_1[...])
    y6_o[...] = y6
    ys6_o[...] = dinv * y6


def _dense7_body(S6, y6, dinv_r, bA_1, h_o):
    dinv = dinv_r[...]
    h_o[...] = _comb(S6, 0, 1, dinv, y6[...], bA_1[...])


def _pad_table(ys, cp):
    n, c = ys.shape
    return jnp.pad(ys, ((0, NPAD - n), (0, cp - c)))


def kernel(x, edge_index, edge_weight, m, f, W1, b1, W1_2, b1_2, W2, b2,
           W2_2, b2_2, W2m, b2m, W2m_1, b2m_1, W2f, b2f, W2f_1, b2f_1,
           WA, bA, WA_1, bA_1):
    pad_idx = N + (jnp.arange(KW - EPW, dtype=jnp.int32) % 16)
    pad_blk = jnp.broadcast_to(pad_idx, (NW, KW - EPW))
    srcw = jnp.concatenate([edge_index[0].reshape(NW, EPW), pad_blk], axis=1)
    dstw = jnp.concatenate([edge_index[1].reshape(NW, EPW), pad_blk], axis=1)
    srcw = srcw.reshape(NW, NCH, CHUNK)
    dstw = dstw.reshape(NW, NCH, CHUNK)

    b1r = b1.reshape(1, -1)
    b1_2r = b1_2.reshape(1, -1)
    b2r = b2.reshape(1, -1)
    b2_2r = b2_2.reshape(1, -1)
    b2mr = b2m.reshape(1, -1)
    b2fr = b2f.reshape(1, -1)
    b2m_1r = b2m_1.reshape(1, -1)
    b2f_1r = b2f_1.reshape(1, -1)
    bAr = bA.reshape(1, -1)
    bA_1r = bA_1.reshape(1, -1)

    degS = _deg(dstw).reshape(2, NPAD, 16)

    dinv, ysx = _tc(
        _prep_body,
        [("S", 16), ("r", 128)],
        [1, 128],
    )(degS, x)
    SxA = _spmm64(_pad_table(ysx[:, 0:64], 64), srcw, dstw)
    SxB = _spmm64(_pad_table(ysx[:, 64:128], 64), srcw, dstw)
    SxA = SxA.reshape(2, NPAD, 64)
    SxB = SxB.reshape(2, NPAD, 64)

    y2m, y2f, ys2c = _tc(
        _dense12_body,
        [("S", 64), ("S", 64), ("r", 128), ("r", 1),
         ("w", (128, 100)), ("w", (128, 100)), ("w", (1, 100)),
         ("w", (1, 100)), ("w", (100, 1)), ("w", (100, 1))],
        [1, 1, 2],
    )(SxA, SxB, x, dinv, W1, W1_2, b1r, b1_2r, W2, W2_2)
    S2 = _spmm16(_pad_table(ys2c, 16), srcw, dstw).reshape(2, NPAD, 16)

    c2m, c2f, ys3 = _tc(
        _dense3_body,
        [("S", 16), ("r", 1), ("r", 1), ("r", 1), ("r", 1), ("r", 1),
         ("w", (1, 1)), ("w", (1, 1))],
        [1, 1, 4],
    )(S2, y2m, y2f, dinv, m, f, b2r, b2_2r)
    S3 = _spmm16(_pad_table(ys3, 16), srcw, dstw).reshape(2, NPAD, 16)

    y4m, y4f, ys4 = _tc(
        _dense4_body,
        [("S", 16), ("r", 1), ("r", 1), ("r", 1), ("r", 1), ("r", 1),
         ("w", (2, 10)), ("w", (2, 10)), ("w", (1, 10)), ("w", (1, 10)),
         ("w", (10, 1)), ("w", (10, 1))],
        [1, 1, 2],
    )(S3, c2m, c2f, m, f, dinv, W2m, W2f, b2mr, b2fr, W2m_1, W2f_1)
    S4 = _spmm16(_pad_table(ys4, 16), srcw, dstw).reshape(2, NPAD, 16)

    hm_br, hf_br, y5, ys5 = _tc(
        _dense5_body,
        [("S", 16), ("r", 1), ("r", 1), ("r", 1),
         ("w", (1, 1)), ("w", (1, 1)), ("w", (2, 10))],
        [1, 1, 10, 10],
    )(S4, y4m, y4f, dinv, b2m_1r, b2f_1r, WA)
    S5 = _spmm16(_pad_table(ys5, 16), srcw, dstw).reshape(2, NPAD, 16)

    y6, ys6 = _tc(
        _dense6_body,
        [("S", 16), ("r", 10), ("r", 1), ("w", (1, 10)), ("w", (10, 1))],
        [1, 1],
    )(S5, y5, dinv, bAr, WA_1)
    S6 = _spmm16(_pad_table(ys6, 16), srcw, dstw).reshape(2, NPAD, 16)

    (h,) = _tc(
        _dense7_body,
        [("S", 16), ("r", 1), ("r", 1), ("w", (1, 1))],
        [1],
    )(S6, y6, dinv, bA_1r)

    return (h, hm_br, hf_br)
